# Initial kernel scaffold; baseline (speedup 1.0000x reference)
import jax
import jax.numpy as jnp
from jax.experimental import pallas as pl


def kernel(adj, x, W1, a_src1, a_dst1, b1, W2, a_src2, a_dst2, b2):
    raise NotImplementedError("write your pallas kernel here")


# speedup vs baseline: 5958.7758x; 5958.7758x over previous
import functools

import jax
import jax.numpy as jnp
from jax.experimental import pallas as pl


def _gat_kernel(adj_ref, x_ref, W1_ref, as1_ref, ad1_ref, b1_ref,
                W2_ref, as2_ref, ad2_ref, b2_ref, out_ref):
    N = adj_ref.shape[1]
    adj = adj_ref[0]
    x = x_ref[0]

    ii = jax.lax.broadcasted_iota(jnp.int32, (N, N), 0)
    jj = jax.lax.broadcasted_iota(jnp.int32, (N, N), 1)
    diag = ii == jj
    live = ((adj != 0) & jnp.logical_not(diag)) | diag
    neg = jnp.float32(-1e30)

    def leaky(v):
        return jnp.where(v >= 0, v, 0.2 * v)

    def masked_attn(as_col, ad_row, h_head):
        logits = jnp.where(live, leaky(as_col + ad_row), neg)
        amax = jnp.max(logits, axis=0, keepdims=True)
        ex = jnp.exp(logits - amax)
        den = jnp.sum(ex, axis=0, keepdims=True) + 1e-16
        p = ex / den
        return jax.lax.dot_general(
            p, h_head, (((0,), (0,)), ((), ())),
            preferred_element_type=jnp.float32)

    h1 = jnp.dot(x, W1_ref[...], preferred_element_type=jnp.float32)
    outs = []
    for hd in range(4):
        h_head = h1[:, hd * 32:(hd + 1) * 32]
        a_s = as1_ref[hd:hd + 1, :]
        a_d = ad1_ref[hd:hd + 1, :]
        as_col = jax.lax.dot_general(
            h_head, a_s, (((1,), (1,)), ((), ())),
            preferred_element_type=jnp.float32)
        ad_row = jax.lax.dot_general(
            a_d, h_head, (((1,), (1,)), ((), ())),
            preferred_element_type=jnp.float32)
        outs.append(masked_attn(as_col, ad_row, h_head))
    g1 = jnp.concatenate(outs, axis=1) + b1_ref[...]
    g1 = jnp.where(g1 > 0, g1, jnp.exp(jnp.minimum(g1, 0.0)) - 1.0)

    h2 = jnp.dot(g1, W2_ref[...], preferred_element_type=jnp.float32)
    as_col2 = jax.lax.dot_general(
        h2, as2_ref[...], (((1,), (1,)), ((), ())),
        preferred_element_type=jnp.float32)
    ad_row2 = jax.lax.dot_general(
        ad2_ref[...], h2, (((1,), (1,)), ((), ())),
        preferred_element_type=jnp.float32)
    out_ref[0] = masked_attn(as_col2, ad_row2, h2) + b2_ref[...]


@jax.jit
def kernel(adj, x, W1, a_src1, a_dst1, b1, W2, a_src2, a_dst2, b2):
    B, N, _ = adj.shape
    in_dim = x.shape[-1]
    heads, hidden = a_src1.shape[1], a_src1.shape[2]
    out_dim = W2.shape[1]

    as1 = a_src1.reshape(heads, hidden)
    ad1 = a_dst1.reshape(heads, hidden)
    as2 = a_src2.reshape(1, out_dim)
    ad2 = a_dst2.reshape(1, out_dim)
    b1r = b1.reshape(1, heads * hidden)
    b2r = b2.reshape(1, out_dim)

    grid = (B,)
    out = pl.pallas_call(
        _gat_kernel,
        grid=grid,
        in_specs=[
            pl.BlockSpec((1, N, N), lambda b: (b, 0, 0)),
            pl.BlockSpec((1, N, in_dim), lambda b: (b, 0, 0)),
            pl.BlockSpec((in_dim, heads * hidden), lambda b: (0, 0)),
            pl.BlockSpec((heads, hidden), lambda b: (0, 0)),
            pl.BlockSpec((heads, hidden), lambda b: (0, 0)),
            pl.BlockSpec((1, heads * hidden), lambda b: (0, 0)),
            pl.BlockSpec((heads * hidden, out_dim), lambda b: (0, 0)),
            pl.BlockSpec((1, out_dim), lambda b: (0, 0)),
            pl.BlockSpec((1, out_dim), lambda b: (0, 0)),
            pl.BlockSpec((1, out_dim), lambda b: (0, 0)),
        ],
        out_specs=pl.BlockSpec((1, N, out_dim), lambda b: (b, 0, 0)),
        out_shape=jax.ShapeDtypeStruct((B, N, out_dim), jnp.float32),
    )(adj, x, W1, as1, ad1, b1r, W2, as2, ad2, b2r)
    return out


# speedup vs baseline: 7014.3045x; 1.1771x over previous
import jax
import jax.numpy as jnp
from jax.experimental import pallas as pl
from jax.experimental.pallas import tpu as pltpu


def _gat_kernel(adj_ref, x_ref, W1_ref, as1_ref, ad1_ref, b1_ref,
                W2_ref, as2_ref, ad2_ref, b2_ref, out_ref):
    N = adj_ref.shape[1]
    adj = adj_ref[0]
    x = x_ref[0]

    ii = jax.lax.broadcasted_iota(jnp.int32, (N, N), 0)
    jj = jax.lax.broadcasted_iota(jnp.int32, (N, N), 1)
    diag = ii == jj
    live = (adj != 0) | diag
    mbias = jnp.where(live, 0.0, -1e30)
    ones_col = jnp.ones((N, 1), dtype=jnp.float32)

    def masked_attn(as_col, ad_row, h_head):
        v = as_col + ad_row
        logits = jnp.maximum(v, 0.2 * v) + mbias
        amax = jnp.max(logits, axis=0, keepdims=True)
        ex = jnp.exp(logits - amax)
        h_aug = jnp.concatenate([h_head, ones_col], axis=1)
        raw = jax.lax.dot_general(
            ex, h_aug, (((0,), (0,)), ((), ())),
            preferred_element_type=jnp.float32)
        den = raw[:, -1:] + 1e-16
        return raw[:, :-1] / den

    h1 = jnp.dot(x, W1_ref[...], preferred_element_type=jnp.float32)
    outs = []
    for hd in range(4):
        h_head = h1[:, hd * 32:(hd + 1) * 32]
        a_s = as1_ref[hd:hd + 1, :]
        a_d = ad1_ref[hd:hd + 1, :]
        as_col = jax.lax.dot_general(
            h_head, a_s, (((1,), (1,)), ((), ())),
            preferred_element_type=jnp.float32)
        ad_row = jax.lax.dot_general(
            a_d, h_head, (((1,), (1,)), ((), ())),
            preferred_element_type=jnp.float32)
        outs.append(masked_attn(as_col, ad_row, h_head))
    g1 = jnp.concatenate(outs, axis=1) + b1_ref[...]
    g1 = jnp.where(g1 > 0, g1, jnp.exp(jnp.minimum(g1, 0.0)) - 1.0)

    h2 = jnp.dot(g1, W2_ref[...], preferred_element_type=jnp.float32)
    as_col2 = jax.lax.dot_general(
        h2, as2_ref[...], (((1,), (1,)), ((), ())),
        preferred_element_type=jnp.float32)
    ad_row2 = jax.lax.dot_general(
        ad2_ref[...], h2, (((1,), (1,)), ((), ())),
        preferred_element_type=jnp.float32)
    out_ref[0] = masked_attn(as_col2, ad_row2, h2) + b2_ref[...]


@jax.jit
def kernel(adj, x, W1, a_src1, a_dst1, b1, W2, a_src2, a_dst2, b2):
    B, N, _ = adj.shape
    in_dim = x.shape[-1]
    heads, hidden = a_src1.shape[1], a_src1.shape[2]
    out_dim = W2.shape[1]

    as1 = a_src1.reshape(heads, hidden)
    ad1 = a_dst1.reshape(heads, hidden)
    as2 = a_src2.reshape(1, out_dim)
    ad2 = a_dst2.reshape(1, out_dim)
    b1r = b1.reshape(1, heads * hidden)
    b2r = b2.reshape(1, out_dim)

    out = pl.pallas_call(
        _gat_kernel,
        grid=(B,),
        in_specs=[
            pl.BlockSpec((1, N, N), lambda b: (b, 0, 0)),
            pl.BlockSpec((1, N, in_dim), lambda b: (b, 0, 0)),
            pl.BlockSpec((in_dim, heads * hidden), lambda b: (0, 0)),
            pl.BlockSpec((heads, hidden), lambda b: (0, 0)),
            pl.BlockSpec((heads, hidden), lambda b: (0, 0)),
            pl.BlockSpec((1, heads * hidden), lambda b: (0, 0)),
            pl.BlockSpec((heads * hidden, out_dim), lambda b: (0, 0)),
            pl.BlockSpec((1, out_dim), lambda b: (0, 0)),
            pl.BlockSpec((1, out_dim), lambda b: (0, 0)),
            pl.BlockSpec((1, out_dim), lambda b: (0, 0)),
        ],
        out_specs=pl.BlockSpec((1, N, out_dim), lambda b: (b, 0, 0)),
        out_shape=jax.ShapeDtypeStruct((B, N, out_dim), jnp.float32),
        compiler_params=pltpu.CompilerParams(
            dimension_semantics=("parallel",)),
    )(adj, x, W1, as1, ad1, b1r, W2, as2, ad2, b2r)
    return out
